# Initial kernel scaffold; baseline (speedup 1.0000x reference)
import jax
import jax.numpy as jnp
from jax.experimental import pallas as pl


def kernel(features, neighbour_distances, edge_index, triplet_idxs, angles, r_ij, r_ik, W_pre, W_tb1, W_tb2, W_3b1, W_3b2, W_post):
    raise NotImplementedError("write your pallas kernel here")


# speedup vs baseline: 4.4537x; 4.4537x over previous
import functools

import jax
import jax.numpy as jnp
import numpy as np
from jax import lax
from jax.experimental import pallas as pl
from jax.experimental.pallas import tpu as pltpu
from jax.experimental.pallas import tpu_sc as plsc

CUTOFF = 5.0
EXP = 16
CW = 80
NTILES = 32


def _ssp(x):
    return jax.nn.softplus(x) - jnp.log(2.0)


def _h_body(f_ref, w_ref, o_ref):
    o_ref[...] = jnp.dot(f_ref[...], w_ref[...], preferred_element_type=jnp.float32)


def _tbw2_body(d_ref, w1_ref, w2_ref, o_ref):
    d = d_ref[0, 0, :]
    centers = lax.broadcasted_iota(jnp.int32, (1, EXP), 1).astype(jnp.float32) * (
        CUTOFF / (EXP - 1))
    gamma = (EXP / CUTOFF) ** 2
    rb = jnp.exp(-gamma * (d[:, None] - centers) ** 2)
    env = 0.5 * (1.0 + jnp.cos(np.pi * d / CUTOFF))
    mask = (d < CUTOFF).astype(jnp.float32)
    rb = rb * (env * mask)[:, None]
    hid = _ssp(jnp.dot(rb, w1_ref[...], preferred_element_type=jnp.float32))
    o_ref[...] = jnp.dot(hid, w2_ref[...], preferred_element_type=jnp.float32)


def _a3_body(rij_ref, rik_ref, ang_ref, w_ref, o_ref):
    rij = rij_ref[0, 0, :]
    rik = rik_ref[0, 0, :]
    ca = jnp.cos(ang_ref[0, 0, :])
    w = w_ref[...]
    pre = (rij[:, None] * w[0][None, :] + rik[:, None] * w[1][None, :]
           + ca[:, None] * w[2][None, :])
    o_ref[...] = _ssp(pre)


def _g_body(s_ref, h_ref, w_ref, o_ref):
    ssum = s_ref[0] + s_ref[1]
    o_ref[...] = h_ref[...] * jnp.dot(ssum, w_ref[...], preferred_element_type=jnp.float32)


def _out_body(h2b_ref, hg_ref, w_ref, o_ref):
    acc = h2b_ref[0] + h2b_ref[1] + hg_ref[0] + hg_ref[1]
    o_ref[...] = jnp.dot(acc, w_ref[...], preferred_element_type=jnp.float32)


def _sc_tri_body(N, T, H,
                 a3_hbm, tidx_hbm, z64_hbm, s64_out,
                 tidx_v, a3_v, s64_sh, sem):
    c = lax.axis_index("c")
    s = lax.axis_index("s")
    wid = s * 2 + c
    rows_n = N // 16
    trows = (T // CW) // NTILES

    pltpu.sync_copy(z64_hbm.at[pl.ds(s * rows_n, rows_n)],
                    s64_sh.at[pl.ds(s * rows_n, rows_n)])
    plsc.subcore_barrier()

    pltpu.sync_copy(tidx_hbm.at[pl.ds(wid * trows, trows)], tidx_v)

    def tri_step(j, _):
        base = (wid * trows + j) * CW
        pltpu.sync_copy(a3_hbm.at[pl.ds(base, CW)], a3_v)
        pltpu.sync_copy(a3_v, s64_sh.at[tidx_v.at[j]], add=True)
        return _
    lax.fori_loop(0, trows, tri_step, 0)

    plsc.subcore_barrier()
    pltpu.sync_copy(s64_sh.at[pl.ds(s * rows_n, rows_n)],
                    s64_out.at[c, pl.ds(s * rows_n, rows_n)])


def _sc_edge_body(N, E, C,
                  tbw2_hbm, dst_hbm, src_hbm, h_hbm, z128_hbm, h2b_out,
                  dst_v, src_v, rows_v, w_v, hout_sh, sem):
    c = lax.axis_index("c")
    s = lax.axis_index("s")
    wid = s * 2 + c
    rows_n = N // 16
    erows = (E // CW) // NTILES

    pltpu.sync_copy(z128_hbm.at[pl.ds(s * rows_n, rows_n)],
                    hout_sh.at[pl.ds(s * rows_n, rows_n)])
    plsc.subcore_barrier()

    pltpu.sync_copy(dst_hbm.at[pl.ds(wid * erows, erows)], dst_v)
    pltpu.sync_copy(src_hbm.at[pl.ds(wid * erows, erows)], src_v)

    def edge_step(j, _):
        base = (wid * erows + j) * CW
        cp = pltpu.async_copy(h_hbm.at[dst_v.at[j]], rows_v, sem)
        pltpu.sync_copy(tbw2_hbm.at[pl.ds(base, CW)], w_v)
        cp.wait()

        def mul_row(i, _):
            for l in range(C // 16):
                sl = pl.ds(l * 16, 16)
                rows_v[i, sl] = rows_v[i, sl] * w_v[i, sl]
            return _
        lax.fori_loop(0, CW, mul_row, 0)
        pltpu.sync_copy(rows_v, hout_sh.at[src_v.at[j]], add=True)
        return _
    lax.fori_loop(0, erows, edge_step, 0)

    plsc.subcore_barrier()
    pltpu.sync_copy(hout_sh.at[pl.ds(s * rows_n, rows_n)],
                    h2b_out.at[c, pl.ds(s * rows_n, rows_n)])


def _sc_gscatter_body(N, C, gp_rows,
                      g_hbm, srcn_hbm, z128_hbm, hg_out,
                      idx_v, g_v, hg_sh, sem):
    c = lax.axis_index("c")
    s = lax.axis_index("s")
    wid = s * 2 + c
    rows_n = N // 16
    rpt = gp_rows // NTILES

    pltpu.sync_copy(z128_hbm.at[pl.ds(s * rows_n, rows_n)],
                    hg_sh.at[pl.ds(s * rows_n, rows_n)])
    plsc.subcore_barrier()

    pltpu.sync_copy(srcn_hbm.at[pl.ds(wid * rpt, rpt)], idx_v)

    def step(j, _):
        base = (wid * rpt + j) * CW
        pltpu.sync_copy(g_hbm.at[pl.ds(base, CW)], g_v)
        pltpu.sync_copy(g_v, hg_sh.at[idx_v.at[j]], add=True)
        return _
    lax.fori_loop(0, rpt, step, 0)

    plsc.subcore_barrier()
    pltpu.sync_copy(hg_sh.at[pl.ds(s * rows_n, rows_n)],
                    hg_out.at[c, pl.ds(s * rows_n, rows_n)])


def kernel(features, neighbour_distances, edge_index, triplet_idxs, angles,
           r_ij, r_ik, W_pre, W_tb1, W_tb2, W_3b1, W_3b2, W_post):
    N, C = features.shape
    E = neighbour_distances.shape[0]
    T = angles.shape[0]
    H = W_3b1.shape[1]
    f32 = jnp.float32

    BN = 2000
    h = pl.pallas_call(
        _h_body,
        grid=(N // BN,),
        in_specs=[pl.BlockSpec((BN, C), lambda i: (i, 0)),
                  pl.BlockSpec((C, C), lambda i: (0, 0))],
        out_specs=pl.BlockSpec((BN, C), lambda i: (i, 0)),
        out_shape=jax.ShapeDtypeStruct((N, C), f32),
    )(features, W_pre)

    BE = 3200
    d3 = neighbour_distances.reshape(E // BE, 1, BE)
    tbw2 = pl.pallas_call(
        _tbw2_body,
        grid=(E // BE,),
        in_specs=[pl.BlockSpec((1, 1, BE), lambda i: (i, 0, 0)),
                  pl.BlockSpec((EXP, EXP), lambda i: (0, 0)),
                  pl.BlockSpec((EXP, C), lambda i: (0, 0))],
        out_specs=pl.BlockSpec((BE, C), lambda i: (i, 0)),
        out_shape=jax.ShapeDtypeStruct((E, C), f32),
    )(d3, W_tb1, W_tb2)

    BT = 4000
    rij3 = r_ij.reshape(T // BT, 1, BT)
    rik3 = r_ik.reshape(T // BT, 1, BT)
    ang3 = angles.reshape(T // BT, 1, BT)
    a3 = pl.pallas_call(
        _a3_body,
        grid=(T // BT,),
        in_specs=[pl.BlockSpec((1, 1, BT), lambda i: (i, 0, 0)),
                  pl.BlockSpec((1, 1, BT), lambda i: (i, 0, 0)),
                  pl.BlockSpec((1, 1, BT), lambda i: (i, 0, 0)),
                  pl.BlockSpec((3, H), lambda i: (0, 0))],
        out_specs=pl.BlockSpec((BT, H), lambda i: (i, 0)),
        out_shape=jax.ShapeDtypeStruct((T, H), f32),
    )(rij3, rik3, ang3, W_3b1)

    tidx2d = triplet_idxs[:, 1].reshape(T // CW, CW)
    dst2d = edge_index[1].reshape(E // CW, CW)
    src2d = edge_index[0].reshape(E // CW, CW)
    z64 = jnp.zeros((N, H), f32)
    z128 = jnp.zeros((N, C), f32)

    mesh = plsc.VectorSubcoreMesh(core_axis_name="c", subcore_axis_name="s",
                                  num_cores=2, num_subcores=16)
    sc_params = pltpu.CompilerParams(use_tc_tiling_on_sc=False)
    sc_tri = functools.partial(
        pl.kernel,
        compiler_params=sc_params,
        out_type=jax.ShapeDtypeStruct((2, N, H), f32),
        mesh=mesh,
        scratch_types=[
            pltpu.VMEM(((T // CW) // NTILES, CW), jnp.int32),
            pltpu.VMEM((CW, H), f32),
            pltpu.VMEM_SHARED((N, H), f32),
            pltpu.SemaphoreType.DMA,
        ],
    )(functools.partial(_sc_tri_body, N, T, H))
    s64p = sc_tri(a3, tidx2d, z64)

    sc_edge = functools.partial(
        pl.kernel,
        compiler_params=sc_params,
        out_type=jax.ShapeDtypeStruct((2, N, C), f32),
        mesh=mesh,
        scratch_types=[
            pltpu.VMEM(((E // CW) // NTILES, CW), jnp.int32),
            pltpu.VMEM(((E // CW) // NTILES, CW), jnp.int32),
            pltpu.VMEM((CW, C), f32),
            pltpu.VMEM((CW, C), f32),
            pltpu.VMEM_SHARED((N, C), f32),
            pltpu.SemaphoreType.DMA,
        ],
    )(functools.partial(_sc_edge_body, N, E, C))
    h2bp = sc_edge(tbw2, dst2d, src2d, h, z128)

    g = pl.pallas_call(
        _g_body,
        grid=(N // BN,),
        in_specs=[pl.BlockSpec((2, BN, H), lambda i: (0, i, 0)),
                  pl.BlockSpec((BN, C), lambda i: (i, 0)),
                  pl.BlockSpec((H, C), lambda i: (0, 0))],
        out_specs=pl.BlockSpec((BN, C), lambda i: (i, 0)),
        out_shape=jax.ShapeDtypeStruct((N, C), f32),
    )(s64p, h, W_3b2)

    gp_rows = ((N // CW) + NTILES - 1) // NTILES * NTILES
    npad = gp_rows * CW
    g_pad = jnp.concatenate([g, jnp.zeros((npad - N, C), f32)], axis=0)
    srcn = jnp.concatenate(
        [edge_index[0, :N], jnp.zeros((npad - N,), jnp.int32)]).reshape(gp_rows, CW)

    sc_g = functools.partial(
        pl.kernel,
        compiler_params=sc_params,
        out_type=jax.ShapeDtypeStruct((2, N, C), f32),
        mesh=mesh,
        scratch_types=[
            pltpu.VMEM((gp_rows // NTILES, CW), jnp.int32),
            pltpu.VMEM((CW, C), f32),
            pltpu.VMEM_SHARED((N, C), f32),
            pltpu.SemaphoreType.DMA,
        ],
    )(functools.partial(_sc_gscatter_body, N, C, gp_rows))
    hgp = sc_g(g_pad, srcn, z128)

    out = pl.pallas_call(
        _out_body,
        grid=(N // BN,),
        in_specs=[pl.BlockSpec((2, BN, C), lambda i: (0, i, 0)),
                  pl.BlockSpec((2, BN, C), lambda i: (0, i, 0)),
                  pl.BlockSpec((C, C), lambda i: (0, 0))],
        out_specs=pl.BlockSpec((BN, C), lambda i: (i, 0)),
        out_shape=jax.ShapeDtypeStruct((N, C), f32),
    )(h2bp, hgp, W_post)
    return out


# speedup vs baseline: 5.1300x; 1.1518x over previous
import functools

import jax
import jax.numpy as jnp
import numpy as np
from jax import lax
from jax.experimental import pallas as pl
from jax.experimental.pallas import tpu as pltpu
from jax.experimental.pallas import tpu_sc as plsc

CUTOFF = 5.0
EXP = 16
CW = 80
NTILES = 32


def _ssp(x):
    return jax.nn.softplus(x) - jnp.log(2.0)


def _h_body(f_ref, w_ref, o_ref):
    o_ref[...] = jnp.dot(f_ref[...], w_ref[...], preferred_element_type=jnp.float32)


def _tbw2_body(d_ref, w1_ref, w2_ref, o_ref):
    d = d_ref[0, 0, :]
    centers = lax.broadcasted_iota(jnp.int32, (1, EXP), 1).astype(jnp.float32) * (
        CUTOFF / (EXP - 1))
    gamma = (EXP / CUTOFF) ** 2
    rb = jnp.exp(-gamma * (d[:, None] - centers) ** 2)
    env = 0.5 * (1.0 + jnp.cos(np.pi * d / CUTOFF))
    mask = (d < CUTOFF).astype(jnp.float32)
    rb = rb * (env * mask)[:, None]
    hid = _ssp(jnp.dot(rb, w1_ref[...], preferred_element_type=jnp.float32))
    o_ref[...] = jnp.dot(hid, w2_ref[...], preferred_element_type=jnp.float32)


def _a3_body(T, BT, rij_ref, rik_ref, ang_ref, w_ref, o_ref):
    rij = rij_ref[0, 0, :]
    rik = rik_ref[0, 0, :]
    ca = jnp.cos(ang_ref[0, 0, :])
    w = w_ref[...]
    pre = (rij[:, None] * w[0][None, :] + rik[:, None] * w[1][None, :]
           + ca[:, None] * w[2][None, :])
    row = pl.program_id(0) * BT + lax.broadcasted_iota(jnp.int32, (BT, 1), 0)
    o_ref[...] = jnp.where(row < T, _ssp(pre), 0.0)


def _g_body(s_ref, h_ref, w_ref, o_ref):
    ssum = s_ref[0] + s_ref[1]
    o_ref[...] = h_ref[...] * jnp.dot(ssum, w_ref[...], preferred_element_type=jnp.float32)


def _out_body(h2b_ref, hg_ref, w_ref, o_ref):
    acc = h2b_ref[0] + h2b_ref[1] + hg_ref[0] + hg_ref[1]
    o_ref[...] = jnp.dot(acc, w_ref[...], preferred_element_type=jnp.float32)


def _sc_tri_body(N, Tp, TCW,
                 a3_hbm, tidx_hbm, z128_hbm, s_out,
                 tidx_v, a3_v, s_sh, sem):
    c = lax.axis_index("c")
    s = lax.axis_index("s")
    wid = s * 2 + c
    rows_n = N // 16
    trows = (Tp // TCW) // NTILES

    pltpu.sync_copy(z128_hbm.at[pl.ds(s * rows_n, rows_n)],
                    s_sh.at[pl.ds(s * rows_n, rows_n)])
    plsc.subcore_barrier()

    pltpu.sync_copy(tidx_hbm.at[pl.ds(wid * trows, trows)], tidx_v)

    def tri_step(j, _):
        base = (wid * trows + j) * TCW
        pltpu.sync_copy(a3_hbm.at[pl.ds(base, TCW)], a3_v)
        pltpu.sync_copy(a3_v, s_sh.at[tidx_v.at[j]], add=True)
        return _
    lax.fori_loop(0, trows, tri_step, 0)

    plsc.subcore_barrier()
    pltpu.sync_copy(s_sh.at[pl.ds(s * rows_n, rows_n)],
                    s_out.at[c, pl.ds(s * rows_n, rows_n)])


def _sc_edge_body(N, E, C,
                  tbw2_hbm, dst_hbm, src_hbm, h_hbm, z128_hbm, h2b_out,
                  dst_v, src_v, rows_v, w_v, hout_sh, sem):
    c = lax.axis_index("c")
    s = lax.axis_index("s")
    wid = s * 2 + c
    rows_n = N // 16
    erows = (E // CW) // NTILES

    pltpu.sync_copy(z128_hbm.at[pl.ds(s * rows_n, rows_n)],
                    hout_sh.at[pl.ds(s * rows_n, rows_n)])
    plsc.subcore_barrier()

    pltpu.sync_copy(dst_hbm.at[pl.ds(wid * erows, erows)], dst_v)
    pltpu.sync_copy(src_hbm.at[pl.ds(wid * erows, erows)], src_v)

    def edge_step(j, _):
        base = (wid * erows + j) * CW
        cp = pltpu.async_copy(h_hbm.at[dst_v.at[j]], rows_v, sem)
        pltpu.sync_copy(tbw2_hbm.at[pl.ds(base, CW)], w_v)
        cp.wait()

        def mul_row(i, _):
            for l in range(C // 16):
                sl = pl.ds(l * 16, 16)
                rows_v[i, sl] = rows_v[i, sl] * w_v[i, sl]
            return _
        lax.fori_loop(0, CW, mul_row, 0)
        pltpu.sync_copy(rows_v, hout_sh.at[src_v.at[j]], add=True)
        return _
    lax.fori_loop(0, erows, edge_step, 0)

    plsc.subcore_barrier()
    pltpu.sync_copy(hout_sh.at[pl.ds(s * rows_n, rows_n)],
                    h2b_out.at[c, pl.ds(s * rows_n, rows_n)])


def _sc_gscatter_body(N, C, gp_rows,
                      g_hbm, srcn_hbm, z128_hbm, hg_out,
                      idx_v, g_v, hg_sh, sem):
    c = lax.axis_index("c")
    s = lax.axis_index("s")
    wid = s * 2 + c
    rows_n = N // 16
    rpt = gp_rows // NTILES

    pltpu.sync_copy(z128_hbm.at[pl.ds(s * rows_n, rows_n)],
                    hg_sh.at[pl.ds(s * rows_n, rows_n)])
    plsc.subcore_barrier()

    pltpu.sync_copy(srcn_hbm.at[pl.ds(wid * rpt, rpt)], idx_v)

    def step(j, _):
        base = (wid * rpt + j) * CW
        pltpu.sync_copy(g_hbm.at[pl.ds(base, CW)], g_v)
        pltpu.sync_copy(g_v, hg_sh.at[idx_v.at[j]], add=True)
        return _
    lax.fori_loop(0, rpt, step, 0)

    plsc.subcore_barrier()
    pltpu.sync_copy(hg_sh.at[pl.ds(s * rows_n, rows_n)],
                    hg_out.at[c, pl.ds(s * rows_n, rows_n)])


def kernel(features, neighbour_distances, edge_index, triplet_idxs, angles,
           r_ij, r_ik, W_pre, W_tb1, W_tb2, W_3b1, W_3b2, W_post):
    N, C = features.shape
    E = neighbour_distances.shape[0]
    T = angles.shape[0]
    H = W_3b1.shape[1]
    f32 = jnp.float32

    BN = 2000
    h = pl.pallas_call(
        _h_body,
        grid=(N // BN,),
        in_specs=[pl.BlockSpec((BN, C), lambda i: (i, 0)),
                  pl.BlockSpec((C, C), lambda i: (0, 0))],
        out_specs=pl.BlockSpec((BN, C), lambda i: (i, 0)),
        out_shape=jax.ShapeDtypeStruct((N, C), f32),
    )(features, W_pre)

    BE = 3200
    d3 = neighbour_distances.reshape(E // BE, 1, BE)
    tbw2 = pl.pallas_call(
        _tbw2_body,
        grid=(E // BE,),
        in_specs=[pl.BlockSpec((1, 1, BE), lambda i: (i, 0, 0)),
                  pl.BlockSpec((EXP, EXP), lambda i: (0, 0)),
                  pl.BlockSpec((EXP, C), lambda i: (0, 0))],
        out_specs=pl.BlockSpec((BE, C), lambda i: (i, 0)),
        out_shape=jax.ShapeDtypeStruct((E, C), f32),
    )(d3, W_tb1, W_tb2)

    TCW = 128
    Tp = ((T // TCW) + NTILES - 1) // NTILES * NTILES * TCW
    BT = 4096
    rij3 = jnp.pad(r_ij, (0, Tp - T)).reshape(Tp // BT, 1, BT)
    rik3 = jnp.pad(r_ik, (0, Tp - T)).reshape(Tp // BT, 1, BT)
    ang3 = jnp.pad(angles, (0, Tp - T)).reshape(Tp // BT, 1, BT)
    W_3b1p = jnp.pad(W_3b1, ((0, 0), (0, C - H)))
    W_3b2p = jnp.pad(W_3b2, ((0, C - H), (0, 0)))
    a3 = pl.pallas_call(
        functools.partial(_a3_body, T, BT),
        grid=(Tp // BT,),
        in_specs=[pl.BlockSpec((1, 1, BT), lambda i: (i, 0, 0)),
                  pl.BlockSpec((1, 1, BT), lambda i: (i, 0, 0)),
                  pl.BlockSpec((1, 1, BT), lambda i: (i, 0, 0)),
                  pl.BlockSpec((3, C), lambda i: (0, 0))],
        out_specs=pl.BlockSpec((BT, C), lambda i: (i, 0)),
        out_shape=jax.ShapeDtypeStruct((Tp, C), f32),
    )(rij3, rik3, ang3, W_3b1p)

    tidx2d = jnp.pad(triplet_idxs[:, 1], (0, Tp - T)).reshape(Tp // TCW, TCW)
    dst2d = edge_index[1].reshape(E // CW, CW)
    src2d = edge_index[0].reshape(E // CW, CW)
    z128 = jnp.zeros((N, C), f32)

    mesh = plsc.VectorSubcoreMesh(core_axis_name="c", subcore_axis_name="s",
                                  num_cores=2, num_subcores=16)
    sc_params = pltpu.CompilerParams(use_tc_tiling_on_sc=False)
    sc_edge = functools.partial(
        pl.kernel,
        compiler_params=sc_params,
        out_type=jax.ShapeDtypeStruct((2, N, C), f32),
        mesh=mesh,
        scratch_types=[
            pltpu.VMEM(((E // CW) // NTILES, CW), jnp.int32),
            pltpu.VMEM(((E // CW) // NTILES, CW), jnp.int32),
            pltpu.VMEM((CW, C), f32),
            pltpu.VMEM((CW, C), f32),
            pltpu.VMEM_SHARED((N, C), f32),
            pltpu.SemaphoreType.DMA,
        ],
    )(functools.partial(_sc_edge_body, N, E, C))
    h2bp = sc_edge(tbw2, dst2d, src2d, h, z128)

    sc_tri = functools.partial(
        pl.kernel,
        compiler_params=sc_params,
        out_type=jax.ShapeDtypeStruct((2, N, C), f32),
        mesh=mesh,
        scratch_types=[
            pltpu.VMEM(((Tp // TCW) // NTILES, TCW), jnp.int32),
            pltpu.VMEM((TCW, C), f32),
            pltpu.VMEM_SHARED((N, C), f32),
            pltpu.SemaphoreType.DMA,
        ],
    )(functools.partial(_sc_tri_body, N, Tp, TCW))
    sp = sc_tri(a3, tidx2d, z128)

    g = pl.pallas_call(
        _g_body,
        grid=(N // BN,),
        in_specs=[pl.BlockSpec((2, BN, C), lambda i: (0, i, 0)),
                  pl.BlockSpec((BN, C), lambda i: (i, 0)),
                  pl.BlockSpec((C, C), lambda i: (0, 0))],
        out_specs=pl.BlockSpec((BN, C), lambda i: (i, 0)),
        out_shape=jax.ShapeDtypeStruct((N, C), f32),
    )(sp, h, W_3b2p)

    gp_rows = ((N // CW) + NTILES - 1) // NTILES * NTILES
    npad = gp_rows * CW
    g_pad = jnp.concatenate([g, jnp.zeros((npad - N, C), f32)], axis=0)
    srcn = jnp.concatenate(
        [edge_index[0, :N], jnp.zeros((npad - N,), jnp.int32)]).reshape(gp_rows, CW)

    sc_g = functools.partial(
        pl.kernel,
        compiler_params=sc_params,
        out_type=jax.ShapeDtypeStruct((2, N, C), f32),
        mesh=mesh,
        scratch_types=[
            pltpu.VMEM((gp_rows // NTILES, CW), jnp.int32),
            pltpu.VMEM((CW, C), f32),
            pltpu.VMEM_SHARED((N, C), f32),
            pltpu.SemaphoreType.DMA,
        ],
    )(functools.partial(_sc_gscatter_body, N, C, gp_rows))
    hgp = sc_g(g_pad, srcn, z128)

    out = pl.pallas_call(
        _out_body,
        grid=(N // BN,),
        in_specs=[pl.BlockSpec((2, BN, C), lambda i: (0, i, 0)),
                  pl.BlockSpec((2, BN, C), lambda i: (0, i, 0)),
                  pl.BlockSpec((C, C), lambda i: (0, 0))],
        out_specs=pl.BlockSpec((BN, C), lambda i: (i, 0)),
        out_shape=jax.ShapeDtypeStruct((N, C), f32),
    )(h2bp, hgp, W_post)
    return out


# speedup vs baseline: 5.5015x; 1.0724x over previous
import functools

import jax
import jax.numpy as jnp
import numpy as np
from jax import lax
from jax.experimental import pallas as pl
from jax.experimental.pallas import tpu as pltpu
from jax.experimental.pallas import tpu_sc as plsc

CUTOFF = 5.0
EXP = 16
CW = 40
NTILES = 32


def _ssp(x):
    return jax.nn.softplus(x) - jnp.log(2.0)


def _h_body(f_ref, w_ref, o_ref):
    o_ref[...] = jnp.dot(f_ref[...], w_ref[...], preferred_element_type=jnp.float32)


def _tbw2_body(d_ref, w1_ref, w2_ref, o_ref):
    d = d_ref[0, 0, :]
    centers = lax.broadcasted_iota(jnp.int32, (1, EXP), 1).astype(jnp.float32) * (
        CUTOFF / (EXP - 1))
    gamma = (EXP / CUTOFF) ** 2
    rb = jnp.exp(-gamma * (d[:, None] - centers) ** 2)
    env = 0.5 * (1.0 + jnp.cos(np.pi * d / CUTOFF))
    mask = (d < CUTOFF).astype(jnp.float32)
    rb = rb * (env * mask)[:, None]
    hid = _ssp(jnp.dot(rb, w1_ref[...], preferred_element_type=jnp.float32))
    o_ref[...] = jnp.dot(hid, w2_ref[...], preferred_element_type=jnp.float32)


def _a3_body(T, BT, rij_ref, rik_ref, ang_ref, w_ref, o_ref):
    rij = rij_ref[0, 0, :]
    rik = rik_ref[0, 0, :]
    ca = jnp.cos(ang_ref[0, 0, :])
    w = w_ref[...]
    pre = (rij[:, None] * w[0][None, :] + rik[:, None] * w[1][None, :]
           + ca[:, None] * w[2][None, :])
    ssp = _ssp(pre)
    row = pl.program_id(0) * BT + lax.broadcasted_iota(jnp.int32, (BT, 1), 0)
    o_ref[...] = jnp.where(row < T, ssp, 0.0)


def _g_body(s_ref, h_ref, w_ref, o_ref):
    ssum = s_ref[0] + s_ref[1]
    o_ref[...] = h_ref[...] * jnp.dot(ssum, w_ref[...], preferred_element_type=jnp.float32)


def _out_body(h2b_ref, hg_ref, w_ref, o_ref):
    acc = h2b_ref[0] + h2b_ref[1] + hg_ref[0] + hg_ref[1]
    o_ref[...] = jnp.dot(acc, w_ref[...], preferred_element_type=jnp.float32)


def _sc_tri_body(N, Tp, TCW,
                 a3_hbm, tidx_hbm, z128_hbm, s_out,
                 tidx_v, a3_v0, a3_v1, s_sh,
                 lsem0, lsem1, ssem0, ssem1):
    c = lax.axis_index("c")
    s = lax.axis_index("s")
    wid = s * 2 + c
    rows_n = N // 16
    trows = (Tp // TCW) // NTILES

    pltpu.sync_copy(z128_hbm.at[pl.ds(s * rows_n, rows_n)],
                    s_sh.at[pl.ds(s * rows_n, rows_n)])
    plsc.subcore_barrier()

    pltpu.sync_copy(tidx_hbm.at[pl.ds(wid * trows, trows)], tidx_v)

    def load_start(j, buf, sem):
        pltpu.async_copy(
            a3_hbm.at[pl.ds((wid * trows + j) * TCW, TCW)], buf, sem)

    def load_wait(j, buf, sem):
        pltpu.make_async_copy(
            a3_hbm.at[pl.ds((wid * trows + j) * TCW, TCW)], buf, sem).wait()

    def scatter_start(j, buf, sem):
        pltpu.async_copy(buf, s_sh.at[tidx_v.at[j]], sem, add=True)

    def scatter_wait(j, buf, sem):
        pltpu.make_async_copy(buf, s_sh.at[tidx_v.at[j]], sem).wait()

    load_start(0, a3_v0, lsem0)
    load_start(1, a3_v1, lsem1)

    def pair(gi, carry):
        j0 = 2 * gi
        j1 = j0 + 1
        load_wait(j0, a3_v0, lsem0)
        scatter_start(j0, a3_v0, ssem0)
        load_wait(j1, a3_v1, lsem1)
        scatter_start(j1, a3_v1, ssem1)
        scatter_wait(j0, a3_v0, ssem0)

        @pl.when(j0 + 2 < trows)
        def _():
            load_start(j0 + 2, a3_v0, lsem0)
        scatter_wait(j1, a3_v1, ssem1)

        @pl.when(j1 + 2 < trows)
        def _():
            load_start(j1 + 2, a3_v1, lsem1)
        return carry
    lax.fori_loop(0, trows // 2, pair, 0)

    plsc.subcore_barrier()
    pltpu.sync_copy(s_sh.at[pl.ds(s * rows_n, rows_n)],
                    s_out.at[c, pl.ds(s * rows_n, rows_n)])


def _sc_edge_body(N, E, C,
                  tbw2_hbm, dst_hbm, src_hbm, h_hbm, z128_hbm, h2b_out,
                  dst_v, src_v, rows_v0, rows_v1, w_v0, w_v1, hout_sh,
                  gsem0, gsem1, wsem0, wsem1, ssem0, ssem1):
    c = lax.axis_index("c")
    s = lax.axis_index("s")
    wid = s * 2 + c
    rows_n = N // 16
    erows = (E // CW) // NTILES

    pltpu.sync_copy(z128_hbm.at[pl.ds(s * rows_n, rows_n)],
                    hout_sh.at[pl.ds(s * rows_n, rows_n)])
    plsc.subcore_barrier()

    pltpu.sync_copy(dst_hbm.at[pl.ds(wid * erows, erows)], dst_v)
    pltpu.sync_copy(src_hbm.at[pl.ds(wid * erows, erows)], src_v)

    def in_start(j, rbuf, wbuf, gsem, wsem):
        pltpu.async_copy(h_hbm.at[dst_v.at[j]], rbuf, gsem)
        pltpu.async_copy(
            tbw2_hbm.at[pl.ds((wid * erows + j) * CW, CW)], wbuf, wsem)

    def in_wait(j, rbuf, wbuf, gsem, wsem):
        pltpu.make_async_copy(h_hbm.at[dst_v.at[j]], rbuf, gsem).wait()
        pltpu.make_async_copy(
            tbw2_hbm.at[pl.ds((wid * erows + j) * CW, CW)], wbuf, wsem).wait()

    def mul(rbuf, wbuf):
        def mul_row(i, c2):
            for l in range(C // 16):
                sl = pl.ds(l * 16, 16)
                rbuf[i, sl] = rbuf[i, sl] * wbuf[i, sl]
            return c2
        lax.fori_loop(0, CW, mul_row, 0)

    in_start(0, rows_v0, w_v0, gsem0, wsem0)
    in_start(1, rows_v1, w_v1, gsem1, wsem1)

    def pair(gi, carry):
        j0 = 2 * gi
        j1 = j0 + 1
        in_wait(j0, rows_v0, w_v0, gsem0, wsem0)
        mul(rows_v0, w_v0)
        pltpu.async_copy(rows_v0, hout_sh.at[src_v.at[j0]], ssem0, add=True)
        in_wait(j1, rows_v1, w_v1, gsem1, wsem1)
        mul(rows_v1, w_v1)
        pltpu.async_copy(rows_v1, hout_sh.at[src_v.at[j1]], ssem1, add=True)
        pltpu.make_async_copy(rows_v0, hout_sh.at[src_v.at[j0]], ssem0).wait()

        @pl.when(j0 + 2 < erows)
        def _():
            in_start(j0 + 2, rows_v0, w_v0, gsem0, wsem0)
        pltpu.make_async_copy(rows_v1, hout_sh.at[src_v.at[j1]], ssem1).wait()

        @pl.when(j1 + 2 < erows)
        def _():
            in_start(j1 + 2, rows_v1, w_v1, gsem1, wsem1)
        return carry
    lax.fori_loop(0, erows // 2, pair, 0)

    plsc.subcore_barrier()
    pltpu.sync_copy(hout_sh.at[pl.ds(s * rows_n, rows_n)],
                    h2b_out.at[c, pl.ds(s * rows_n, rows_n)])


def _sc_gscatter_body(N, C, gp_rows,
                      g_hbm, srcn_hbm, z128_hbm, hg_out,
                      idx_v, g_v, hg_sh, sem):
    c = lax.axis_index("c")
    s = lax.axis_index("s")
    wid = s * 2 + c
    rows_n = N // 16
    rpt = gp_rows // NTILES

    pltpu.sync_copy(z128_hbm.at[pl.ds(s * rows_n, rows_n)],
                    hg_sh.at[pl.ds(s * rows_n, rows_n)])
    plsc.subcore_barrier()

    pltpu.sync_copy(srcn_hbm.at[pl.ds(wid * rpt, rpt)], idx_v)

    def step(j, _):
        base = (wid * rpt + j) * CW
        pltpu.sync_copy(g_hbm.at[pl.ds(base, CW)], g_v)
        pltpu.sync_copy(g_v, hg_sh.at[idx_v.at[j]], add=True)
        return _
    lax.fori_loop(0, rpt, step, 0)

    plsc.subcore_barrier()
    pltpu.sync_copy(hg_sh.at[pl.ds(s * rows_n, rows_n)],
                    hg_out.at[c, pl.ds(s * rows_n, rows_n)])


def kernel(features, neighbour_distances, edge_index, triplet_idxs, angles,
           r_ij, r_ik, W_pre, W_tb1, W_tb2, W_3b1, W_3b2, W_post):
    N, C = features.shape
    E = neighbour_distances.shape[0]
    T = angles.shape[0]
    H = W_3b1.shape[1]
    f32 = jnp.float32

    BN = 2000
    h = pl.pallas_call(
        _h_body,
        grid=(N // BN,),
        in_specs=[pl.BlockSpec((BN, C), lambda i: (i, 0)),
                  pl.BlockSpec((C, C), lambda i: (0, 0))],
        out_specs=pl.BlockSpec((BN, C), lambda i: (i, 0)),
        out_shape=jax.ShapeDtypeStruct((N, C), f32),
    )(features, W_pre)

    BE = 3200
    d3 = neighbour_distances.reshape(E // BE, 1, BE)
    tbw2 = pl.pallas_call(
        _tbw2_body,
        grid=(E // BE,),
        in_specs=[pl.BlockSpec((1, 1, BE), lambda i: (i, 0, 0)),
                  pl.BlockSpec((EXP, EXP), lambda i: (0, 0)),
                  pl.BlockSpec((EXP, C), lambda i: (0, 0))],
        out_specs=pl.BlockSpec((BE, C), lambda i: (i, 0)),
        out_shape=jax.ShapeDtypeStruct((E, C), f32),
    )(d3, W_tb1, W_tb2)

    TCW = 64
    BT = 4096
    Tp = -(-T // (NTILES * TCW * (BT // TCW))) * (NTILES * TCW * (BT // TCW))
    rij3 = jnp.pad(r_ij, (0, Tp - T)).reshape(Tp // BT, 1, BT)
    rik3 = jnp.pad(r_ik, (0, Tp - T)).reshape(Tp // BT, 1, BT)
    ang3 = jnp.pad(angles, (0, Tp - T)).reshape(Tp // BT, 1, BT)
    W_3b1p = jnp.pad(W_3b1, ((0, 0), (0, C - H)))
    W_3b2p = jnp.pad(W_3b2, ((0, C - H), (0, 0)))
    a3 = pl.pallas_call(
        functools.partial(_a3_body, T, BT),
        grid=(Tp // BT,),
        in_specs=[pl.BlockSpec((1, 1, BT), lambda i: (i, 0, 0)),
                  pl.BlockSpec((1, 1, BT), lambda i: (i, 0, 0)),
                  pl.BlockSpec((1, 1, BT), lambda i: (i, 0, 0)),
                  pl.BlockSpec((3, C), lambda i: (0, 0))],
        out_specs=pl.BlockSpec((BT, C), lambda i: (i, 0)),
        out_shape=jax.ShapeDtypeStruct((Tp, C), f32),
    )(rij3, rik3, ang3, W_3b1p)

    tidx2d = jnp.pad(triplet_idxs[:, 1], (0, Tp - T)).reshape(Tp // TCW, TCW)
    dst2d = edge_index[1].reshape(E // CW, CW)
    src2d = edge_index[0].reshape(E // CW, CW)
    z128 = jnp.zeros((N, C), f32)

    mesh = plsc.VectorSubcoreMesh(core_axis_name="c", subcore_axis_name="s",
                                  num_cores=2, num_subcores=16)
    sc_params = pltpu.CompilerParams(use_tc_tiling_on_sc=False)
    sc_edge = functools.partial(
        pl.kernel,
        compiler_params=sc_params,
        out_type=jax.ShapeDtypeStruct((2, N, C), f32),
        mesh=mesh,
        scratch_types=[
            pltpu.VMEM(((E // CW) // NTILES, CW), jnp.int32),
            pltpu.VMEM(((E // CW) // NTILES, CW), jnp.int32),
            pltpu.VMEM((CW, C), f32),
            pltpu.VMEM((CW, C), f32),
            pltpu.VMEM((CW, C), f32),
            pltpu.VMEM((CW, C), f32),
            pltpu.VMEM_SHARED((N, C), f32),
            pltpu.SemaphoreType.DMA,
            pltpu.SemaphoreType.DMA,
            pltpu.SemaphoreType.DMA,
            pltpu.SemaphoreType.DMA,
            pltpu.SemaphoreType.DMA,
            pltpu.SemaphoreType.DMA,
        ],
    )(functools.partial(_sc_edge_body, N, E, C))
    h2bp = sc_edge(tbw2, dst2d, src2d, h, z128)

    sc_tri = functools.partial(
        pl.kernel,
        compiler_params=sc_params,
        out_type=jax.ShapeDtypeStruct((2, N, C), f32),
        mesh=mesh,
        scratch_types=[
            pltpu.VMEM(((Tp // TCW) // NTILES, TCW), jnp.int32),
            pltpu.VMEM((TCW, C), f32),
            pltpu.VMEM((TCW, C), f32),
            pltpu.VMEM_SHARED((N, C), f32),
            pltpu.SemaphoreType.DMA,
            pltpu.SemaphoreType.DMA,
            pltpu.SemaphoreType.DMA,
            pltpu.SemaphoreType.DMA,
        ],
    )(functools.partial(_sc_tri_body, N, Tp, TCW))
    sp = sc_tri(a3, tidx2d, z128)

    g = pl.pallas_call(
        _g_body,
        grid=(N // BN,),
        in_specs=[pl.BlockSpec((2, BN, C), lambda i: (0, i, 0)),
                  pl.BlockSpec((BN, C), lambda i: (i, 0)),
                  pl.BlockSpec((C, C), lambda i: (0, 0))],
        out_specs=pl.BlockSpec((BN, C), lambda i: (i, 0)),
        out_shape=jax.ShapeDtypeStruct((N, C), f32),
    )(sp, h, W_3b2p)

    gp_rows = ((N // CW) + NTILES - 1) // NTILES * NTILES
    npad = gp_rows * CW
    g_pad = jnp.concatenate([g, jnp.zeros((npad - N, C), f32)], axis=0)
    srcn = jnp.concatenate(
        [edge_index[0, :N], jnp.zeros((npad - N,), jnp.int32)]).reshape(gp_rows, CW)

    sc_g = functools.partial(
        pl.kernel,
        compiler_params=sc_params,
        out_type=jax.ShapeDtypeStruct((2, N, C), f32),
        mesh=mesh,
        scratch_types=[
            pltpu.VMEM((gp_rows // NTILES, CW), jnp.int32),
            pltpu.VMEM((CW, C), f32),
            pltpu.VMEM_SHARED((N, C), f32),
            pltpu.SemaphoreType.DMA,
        ],
    )(functools.partial(_sc_gscatter_body, N, C, gp_rows))
    hgp = sc_g(g_pad, srcn, z128)

    out = pl.pallas_call(
        _out_body,
        grid=(N // BN,),
        in_specs=[pl.BlockSpec((2, BN, C), lambda i: (0, i, 0)),
                  pl.BlockSpec((2, BN, C), lambda i: (0, i, 0)),
                  pl.BlockSpec((C, C), lambda i: (0, 0))],
        out_specs=pl.BlockSpec((BN, C), lambda i: (i, 0)),
        out_shape=jax.ShapeDtypeStruct((N, C), f32),
    )(h2bp, hgp, W_post)
    return out
